# Initial kernel scaffold; baseline (speedup 1.0000x reference)
import jax
import jax.numpy as jnp
from jax.experimental import pallas as pl


def kernel(inputs, h0, v, w, Wm, bm, Wk, Wr, bias):
    raise NotImplementedError("write your pallas kernel here")


# speedup vs baseline: 160.3897x; 160.3897x over previous
import functools

import jax
import jax.numpy as jnp
from jax import lax
from jax.experimental import pallas as pl
from jax.experimental.pallas import tpu as pltpu
from jax.experimental.pallas import tpu_sc as plsc

NC = 2
NS = 16
CHUNK = 128
SB = 8

_SELU_ALPHA = 1.6732632423543772
_SELU_SCALE = 1.0507009873554805


def _sigmoid(x):
    return 1.0 / (1.0 + jnp.exp(-x))


def _msg_body(h_ref, wm_ref, bm_ref, o_ref):
    x = jnp.dot(h_ref[...], wm_ref[...], preferred_element_type=jnp.float32)
    x = x + bm_ref[...]
    o_ref[...] = _SELU_SCALE * jnp.where(
        x > 0, x, _SELU_ALPHA * (jnp.exp(x) - 1.0))


def _gru_body(x_ref, m_ref, h_ref, wk1_ref, wk2_ref, wr_ref, b0_ref, b1_ref,
              o_ref, *, units):
    h = h_ref[...]
    mx = (jnp.dot(x_ref[...], wk1_ref[...], preferred_element_type=jnp.float32)
          + jnp.dot(m_ref[...], wk2_ref[...], preferred_element_type=jnp.float32)
          + b0_ref[...])
    mi = jnp.dot(h, wr_ref[...], preferred_element_type=jnp.float32) + b1_ref[...]
    U = units
    z = _sigmoid(mx[:, :U] + mi[:, :U])
    r = _sigmoid(mx[:, U:2 * U] + mi[:, U:2 * U])
    hh = jnp.tanh(mx[:, 2 * U:] + r * mi[:, 2 * U:])
    o_ref[...] = z * h + (1.0 - z) * hh


def _make_sc_seg_sum(links, mu, nch_pad):
    cpt = nch_pad // NS
    ngroups = cpt // SB
    rows_per_tile = -(-links // (NS * CHUNK)) * CHUNK
    links_pad = rows_per_tile * NS
    nzero = rows_per_tile // CHUNK
    acc_rows = links_pad

    mesh = plsc.VectorSubcoreMesh(core_axis_name="c", subcore_axis_name="s",
                                  num_cores=NC, num_subcores=NS)

    @functools.partial(
        pl.kernel,
        out_type=jax.ShapeDtypeStruct((NC * links_pad, mu), jnp.float32),
        mesh=mesh,
        scratch_types=[
            pltpu.VMEM_SHARED((acc_rows, mu), jnp.float32),
            pltpu.VMEM((SB, CHUNK), jnp.int32),
            pltpu.VMEM((SB, CHUNK), jnp.int32),
            pltpu.VMEM((SB, CHUNK, mu), jnp.float32),
            pltpu.SemaphoreType.DMA,
        ],
        compiler_params=pltpu.CompilerParams(use_tc_tiling_on_sc=False),
    )
    def sc_fn(msg_hbm, voff_hbm, w_hbm, out_hbm, acc, idxv, idxw, rows, gsem):
        cid = lax.axis_index("c")
        sid = lax.axis_index("s")

        def zstore(i, carry):
            rows[0, i, :] = jnp.zeros((mu,), jnp.float32)
            return carry
        lax.fori_loop(0, CHUNK, zstore, 0)

        base = sid * rows_per_tile
        zcopy = rows.at[0]
        for k in range(nzero):
            pltpu.sync_copy(zcopy, acc.at[pl.ds(base + k * CHUNK, CHUNK)])

        plsc.subcore_barrier()

        def group(g, carry):
            row0 = cid * nch_pad + sid * cpt + g * SB
            wrow0 = sid * cpt + g * SB
            pltpu.sync_copy(voff_hbm.at[pl.ds(row0, SB)], idxv)
            pltpu.sync_copy(w_hbm.at[pl.ds(wrow0, SB)], idxw)
            descs = [
                pltpu.async_copy(msg_hbm.at[idxv.at[j]], rows.at[j], gsem)
                for j in range(SB)
            ]
            for d in descs:
                d.wait()
            for j in range(SB):
                pltpu.sync_copy(rows.at[j], acc.at[idxw.at[j]], add=True)
            return carry
        lax.fori_loop(0, ngroups, group, 0)

        plsc.subcore_barrier()

        src0 = sid * rows_per_tile
        dst0 = cid * links_pad + sid * rows_per_tile
        pltpu.sync_copy(acc.at[pl.ds(src0, rows_per_tile)],
                        out_hbm.at[pl.ds(dst0, rows_per_tile)])

    return sc_fn


def kernel(inputs, h0, v, w, Wm, bm, Wk, Wr, bias):
    B, LINKS, FEAT = inputs.shape
    UNITS = h0.shape[2]
    MU = Wm.shape[1]
    E = v.shape[0]
    NR = B * LINKS
    RB = 1000

    msg2d = pl.pallas_call(
        _msg_body,
        grid=(NR // RB,),
        in_specs=[pl.BlockSpec((RB, UNITS), lambda i: (i, 0)),
                  pl.BlockSpec((UNITS, MU), lambda i: (0, 0)),
                  pl.BlockSpec((1, MU), lambda i: (0, 0))],
        out_specs=pl.BlockSpec((RB, MU), lambda i: (i, 0)),
        out_shape=jax.ShapeDtypeStruct((NR, MU), jnp.float32),
    )(h0.reshape(NR, UNITS), Wm, bm.reshape(1, MU))

    NCH_PAD = -(-E // (CHUNK * NS * SB)) * (NS * SB)
    EP = NCH_PAD * CHUNK
    pad = EP - E
    vp = jnp.concatenate([v, jnp.zeros((pad,), jnp.int32)])
    wp = jnp.concatenate([w, jnp.full((pad,), LINKS, jnp.int32)])
    offs = (jnp.arange(B, dtype=jnp.int32) * LINKS)[:, None]
    voff = (vp[None, :] + offs).reshape(B * NCH_PAD, CHUNK)
    w2d = wp.reshape(NCH_PAD, CHUNK)
    m2d_full = _make_sc_seg_sum(LINKS, MU, NCH_PAD)(msg2d, voff, w2d)
    LP = m2d_full.shape[0] // B
    m2d = m2d_full.reshape(B, LP, MU)[:, :LINKS, :].reshape(NR, MU)

    out2d = pl.pallas_call(
        functools.partial(_gru_body, units=UNITS),
        grid=(NR // RB,),
        in_specs=[pl.BlockSpec((RB, FEAT), lambda i: (i, 0)),
                  pl.BlockSpec((RB, MU), lambda i: (i, 0)),
                  pl.BlockSpec((RB, UNITS), lambda i: (i, 0)),
                  pl.BlockSpec((FEAT, 3 * UNITS), lambda i: (0, 0)),
                  pl.BlockSpec((MU, 3 * UNITS), lambda i: (0, 0)),
                  pl.BlockSpec((UNITS, 3 * UNITS), lambda i: (0, 0)),
                  pl.BlockSpec((1, 3 * UNITS), lambda i: (0, 0)),
                  pl.BlockSpec((1, 3 * UNITS), lambda i: (0, 0))],
        out_specs=pl.BlockSpec((RB, UNITS), lambda i: (i, 0)),
        out_shape=jax.ShapeDtypeStruct((NR, UNITS), jnp.float32),
    )(inputs.reshape(NR, FEAT), m2d, h0.reshape(NR, UNITS),
      Wk[:FEAT], Wk[FEAT:], Wr, bias[0:1], bias[1:2])

    return out2d.reshape(B, LINKS, UNITS)


# speedup vs baseline: 221.4570x; 1.3807x over previous
import functools

import jax
import jax.numpy as jnp
from jax import lax
from jax.experimental import pallas as pl
from jax.experimental.pallas import tpu as pltpu
from jax.experimental.pallas import tpu_sc as plsc

NC = 2
NS = 16
CHUNK = 128
SB = 5

_SELU_ALPHA = 1.6732632423543772
_SELU_SCALE = 1.0507009873554805


def _sigmoid(x):
    return 1.0 / (1.0 + jnp.exp(-x))


def _msg_body(h_ref, wm_ref, bm_ref, o_ref):
    x = jnp.dot(h_ref[0], wm_ref[...], preferred_element_type=jnp.float32)
    x = x + bm_ref[...]
    o_ref[0] = _SELU_SCALE * jnp.where(
        x > 0, x, _SELU_ALPHA * (jnp.exp(x) - 1.0))


def _gru_body(x_ref, m_ref, h_ref, wk1_ref, wk2_ref, wr_ref, b0_ref, b1_ref,
              o_ref, *, units):
    h = h_ref[0]
    mx = (jnp.dot(x_ref[0], wk1_ref[...], preferred_element_type=jnp.float32)
          + jnp.dot(m_ref[0], wk2_ref[...], preferred_element_type=jnp.float32)
          + b0_ref[...])
    mi = jnp.dot(h, wr_ref[...], preferred_element_type=jnp.float32) + b1_ref[...]
    U = units
    z = _sigmoid(mx[:, :U] + mi[:, :U])
    r = _sigmoid(mx[:, U:2 * U] + mi[:, U:2 * U])
    hh = jnp.tanh(mx[:, 2 * U:] + r * mi[:, 2 * U:])
    o_ref[0] = z * h + (1.0 - z) * hh


def _make_sc_seg_sum(links, mu, nch):
    cpt = -(-nch // NS)
    PAIR = 2 * SB
    rows_per_tile = -(-links // (NS * CHUNK)) * CHUNK
    links_pad = rows_per_tile * NS
    nzero = rows_per_tile // CHUNK
    wpt = links // NS

    mesh = plsc.VectorSubcoreMesh(core_axis_name="c", subcore_axis_name="s",
                                  num_cores=NC, num_subcores=NS)

    @functools.partial(
        pl.kernel,
        out_type=jax.ShapeDtypeStruct((NC, links, mu), jnp.float32),
        mesh=mesh,
        scratch_types=[
            pltpu.VMEM_SHARED((links_pad, mu), jnp.float32),
            pltpu.VMEM((SB, CHUNK), jnp.int32),
            pltpu.VMEM((SB, CHUNK), jnp.int32),
            pltpu.VMEM((SB, CHUNK), jnp.int32),
            pltpu.VMEM((SB, CHUNK), jnp.int32),
            pltpu.VMEM((SB, CHUNK, mu), jnp.float32),
            pltpu.VMEM((SB, CHUNK, mu), jnp.float32),
            pltpu.SemaphoreType.DMA,
            pltpu.SemaphoreType.DMA,
            pltpu.SemaphoreType.DMA,
        ],
        compiler_params=pltpu.CompilerParams(use_tc_tiling_on_sc=False),
    )
    def sc_fn(msg_hbm, v_hbm, w_hbm, out_hbm, acc, iv0, iw0, iv1, iw1, r0, r1,
              g0, g1, ss):
        cid = lax.axis_index("c")
        sid = lax.axis_index("s")
        msg_b = msg_hbm.at[cid]
        ivs, iws, rws, gsems = (iv0, iv1), (iw0, iw1), (r0, r1), (g0, g1)

        def zstore(i, carry):
            r0[0, i, :] = jnp.zeros((mu,), jnp.float32)
            return carry
        lax.fori_loop(0, CHUNK, zstore, 0)
        zsrc = r0.at[0]
        base = sid * rows_per_tile
        for k in range(nzero):
            pltpu.sync_copy(zsrc, acc.at[pl.ds(base + k * CHUNK, CHUNK)])

        plsc.subcore_barrier()

        row_base = sid * cpt
        n_t = jnp.maximum(jnp.minimum(cpt, nch - row_base), 0)
        nbody = n_t // PAIR

        def body(p, carry):
            ra = row_base + p * PAIR
            gd = []
            for q in range(2):
                rq = ra + q * SB
                pltpu.sync_copy(v_hbm.at[pl.ds(rq, SB)], ivs[q])
                pltpu.sync_copy(w_hbm.at[pl.ds(rq, SB)], iws[q])
                gd.append([
                    pltpu.async_copy(msg_b.at[ivs[q].at[j]], rws[q].at[j],
                                     gsems[q])
                    for j in range(SB)
                ])
            sd = []
            for q in range(2):
                for d in gd[q]:
                    d.wait()
                sd += [
                    pltpu.async_copy(rws[q].at[j], acc.at[iws[q].at[j]], ss,
                                     add=True)
                    for j in range(SB)
                ]
            for d in sd:
                d.wait()
            return carry
        lax.fori_loop(0, nbody, body, 0)

        ntail = n_t - nbody * PAIR

        def tail(t, carry):
            r = row_base + nbody * PAIR + t
            pltpu.sync_copy(v_hbm.at[pl.ds(r, 1)], iv0.at[pl.ds(0, 1)])
            pltpu.sync_copy(w_hbm.at[pl.ds(r, 1)], iw0.at[pl.ds(0, 1)])
            pltpu.async_copy(msg_b.at[iv0.at[0]], r0.at[0], g0).wait()
            pltpu.sync_copy(r0.at[0], acc.at[iw0.at[0]], add=True)
            return carry
        lax.fori_loop(0, ntail, tail, 0)

        plsc.subcore_barrier()

        pltpu.sync_copy(acc.at[pl.ds(sid * wpt, wpt)],
                        out_hbm.at[cid].at[pl.ds(sid * wpt, wpt)])

    return sc_fn


def kernel(inputs, h0, v, w, Wm, bm, Wk, Wr, bias):
    B, LINKS, FEAT = inputs.shape
    UNITS = h0.shape[2]
    MU = Wm.shape[1]
    E = v.shape[0]
    RB = 2000
    NB = LINKS // RB

    msg3 = pl.pallas_call(
        _msg_body,
        grid=(B, NB),
        in_specs=[pl.BlockSpec((1, RB, UNITS), lambda b, i: (b, i, 0)),
                  pl.BlockSpec((UNITS, MU), lambda b, i: (0, 0)),
                  pl.BlockSpec((1, MU), lambda b, i: (0, 0))],
        out_specs=pl.BlockSpec((1, RB, MU), lambda b, i: (b, i, 0)),
        out_shape=jax.ShapeDtypeStruct((B, LINKS, MU), jnp.float32),
    )(h0, Wm, bm.reshape(1, MU))

    NCH = E // CHUNK
    v2d = v.reshape(NCH, CHUNK)
    w2d = w.reshape(NCH, CHUNK)
    m3 = _make_sc_seg_sum(LINKS, MU, NCH)(msg3, v2d, w2d)

    out3 = pl.pallas_call(
        functools.partial(_gru_body, units=UNITS),
        grid=(B, NB),
        in_specs=[pl.BlockSpec((1, RB, FEAT), lambda b, i: (b, i, 0)),
                  pl.BlockSpec((1, RB, MU), lambda b, i: (b, i, 0)),
                  pl.BlockSpec((1, RB, UNITS), lambda b, i: (b, i, 0)),
                  pl.BlockSpec((FEAT, 3 * UNITS), lambda b, i: (0, 0)),
                  pl.BlockSpec((MU, 3 * UNITS), lambda b, i: (0, 0)),
                  pl.BlockSpec((UNITS, 3 * UNITS), lambda b, i: (0, 0)),
                  pl.BlockSpec((1, 3 * UNITS), lambda b, i: (0, 0)),
                  pl.BlockSpec((1, 3 * UNITS), lambda b, i: (0, 0))],
        out_specs=pl.BlockSpec((1, RB, UNITS), lambda b, i: (b, i, 0)),
        out_shape=jax.ShapeDtypeStruct((B, LINKS, UNITS), jnp.float32),
    )(inputs, m3, h0, Wk[:FEAT], Wk[FEAT:], Wr, bias[0:1], bias[1:2])

    return out3
